# Initial kernel scaffold; baseline (speedup 1.0000x reference)
import jax
import jax.numpy as jnp
from jax.experimental import pallas as pl


def kernel(x, gate_W, gate_b, expert_W):
    raise NotImplementedError("write your pallas kernel here")


# speedup vs baseline: 5.0760x; 5.0760x over previous
import functools

import jax
import jax.numpy as jnp
from jax.experimental import pallas as pl

N = 32768
D = 768
E = 8
M = 8


def _moe_body(x_ref, wcat_ref, gate_b_ref, out_ref):
    x = x_ref[...]
    y = jnp.dot(x, wcat_ref[...],
                preferred_element_type=jnp.float32)
    g = y[:, :E] + gate_b_ref[...]
    aeo = y[:, E:]

    bn = g.shape[0]
    lane_e = jax.lax.broadcasted_iota(jnp.int32, (bn, E), 1)
    m1 = jnp.max(g, axis=1, keepdims=True)
    i1 = jnp.min(jnp.where(g == m1, lane_e, E), axis=1, keepdims=True)
    g2 = jnp.where(lane_e == i1, -jnp.inf, g)
    m2 = jnp.max(g2, axis=1, keepdims=True)
    i2 = jnp.min(jnp.where(g2 == m2, lane_e, E), axis=1, keepdims=True)

    t = jnp.exp(m2 - m1)
    inv = 1.0 / (1.0 + t)
    w1 = inv
    w2 = t * inv

    lane_em = jax.lax.broadcasted_iota(jnp.int32, (bn, E * M), 1)
    grp = jax.lax.div(lane_em, M)
    w_em = jnp.where(grp == i1, w1, 0.0) + jnp.where(grp == i2, w2, 0.0)
    weighted = aeo * w_em

    ri = jax.lax.broadcasted_iota(jnp.int32, (E * M, M), 0)
    rj = jax.lax.broadcasted_iota(jnp.int32, (E * M, M), 1)
    r = (jax.lax.rem(ri, M) == rj).astype(jnp.float32)
    out_ref[...] = jnp.dot(weighted, r,
                           preferred_element_type=jnp.float32)


@functools.partial(jax.jit, static_argnames=("block_n",))
def _moe_forward(x, wcat, gate_b2d, block_n=2048):
    grid = (N // block_n,)
    return pl.pallas_call(
        _moe_body,
        grid=grid,
        in_specs=[
            pl.BlockSpec((block_n, D), lambda i: (i, 0)),
            pl.BlockSpec((D, E + E * M), lambda i: (0, 0)),
            pl.BlockSpec((1, E), lambda i: (0, 0)),
        ],
        out_specs=pl.BlockSpec((block_n, M), lambda i: (i, 0)),
        out_shape=jax.ShapeDtypeStruct((N, M), jnp.float32),
    )(x, wcat, gate_b2d)


def kernel(x, gate_W, gate_b, expert_W):
    w_experts = jnp.transpose(expert_W, (1, 0, 2)).reshape(D, E * M)
    wcat = jnp.concatenate([gate_W, w_experts], axis=1)
    return _moe_forward(x, wcat, gate_b.reshape(1, E))
